# baseline (speedup 1.0000x reference)
import functools

import jax
import jax.numpy as jnp
from jax import lax
from jax.experimental import pallas as pl
from jax.experimental.pallas import tpu as pltpu
from jax.experimental.pallas import tpu_sc as plsc

N = 10000
F = 128
E = 320000
NW = 32
EPW = E // NW
CH = 80
NCH = EPW // CH
SCH = 25
NSC = NCH // SCH
RPT = 624
TAIL_OFF = RPT * 16
TAIL = N - TAIL_OFF


def _sc_segment_sum(h, src3, dst3, zeros):
  mesh = plsc.VectorSubcoreMesh(core_axis_name="c", subcore_axis_name="s")

  @functools.partial(
      pl.kernel,
      out_type=jax.ShapeDtypeStruct((2, N, F), jnp.float32),
      mesh=mesh,
      scratch_types=[
          pltpu.VMEM((SCH, CH), jnp.int32),
          pltpu.VMEM((SCH, CH), jnp.int32),
          pltpu.VMEM((CH, F), jnp.float32),
          pltpu.VMEM((CH, F), jnp.float32),
          pltpu.VMEM((CH, F), jnp.float32),
          pltpu.VMEM((CH, F), jnp.float32),
          pltpu.VMEM_SHARED((N, F), jnp.float32),
          pltpu.SemaphoreType.DMA,
          pltpu.SemaphoreType.DMA,
          pltpu.SemaphoreType.DMA,
          pltpu.SemaphoreType.DMA,
          pltpu.SemaphoreType.DMA,
          pltpu.SemaphoreType.DMA,
          pltpu.SemaphoreType.DMA,
          pltpu.SemaphoreType.DMA,
      ],
  )
  def k(h_hbm, src_hbm, dst_hbm, z_hbm, out_hbm, src_v, dst_v, rows0_v,
        rows1_v, rows2_v, rows3_v, agg_s, sem0, sem1, sem2, sem3,
        ss0, ss1, ss2, ss3):
    cid = lax.axis_index("c")
    sid = lax.axis_index("s")
    wid = cid * 16 + sid
    pltpu.sync_copy(z_hbm.at[pl.ds(sid * RPT, RPT)],
                    agg_s.at[pl.ds(sid * RPT, RPT)])

    @pl.when(sid == 15)
    def _zero_tail():
      pltpu.sync_copy(z_hbm.at[pl.ds(TAIL_OFF, TAIL)],
                      agg_s.at[pl.ds(TAIL_OFF, TAIL)])

    plsc.subcore_barrier()

    def gather(j, buf, sem):
      return pltpu.async_copy(h_hbm.at[src_v.at[j]], buf, sem)

    def wait_gather(j, buf, sem):
      pltpu.make_async_copy(h_hbm.at[src_v.at[j]], buf, sem).wait()

    def scatter(j, buf):
      pltpu.sync_copy(buf, agg_s.at[dst_v.at[j]], add=True)

    def fire_sc(j, buf, sem):
      pltpu.async_copy(buf, agg_s.at[dst_v.at[j]], sem, add=True)

    def wait_sc(j, buf, sem):
      pltpu.make_async_copy(buf, agg_s.at[dst_v.at[j]], sem).wait()

    def superchunk(s, carry):
      pltpu.sync_copy(src_hbm.at[wid, s], src_v)
      pltpu.sync_copy(dst_hbm.at[wid, s], dst_v)
      gather(0, rows0_v, sem0)
      gather(1, rows1_v, sem1)
      wait_gather(0, rows0_v, sem0)
      fire_sc(0, rows0_v, ss0)
      gather(2, rows2_v, sem2)
      wait_gather(1, rows1_v, sem1)
      fire_sc(1, rows1_v, ss1)
      gather(3, rows3_v, sem3)

      def body(i, c):
        j = 4 * i + 2
        wait_gather(j, rows2_v, sem2)
        fire_sc(j, rows2_v, ss2)
        wait_sc(j - 2, rows0_v, ss0)
        gather(j + 2, rows0_v, sem0)
        wait_gather(j + 1, rows3_v, sem3)
        fire_sc(j + 1, rows3_v, ss3)
        wait_sc(j - 1, rows1_v, ss1)
        gather(j + 3, rows1_v, sem1)
        wait_gather(j + 2, rows0_v, sem0)
        fire_sc(j + 2, rows0_v, ss0)
        wait_sc(j, rows2_v, ss2)
        gather(j + 4, rows2_v, sem2)
        wait_gather(j + 3, rows1_v, sem1)
        fire_sc(j + 3, rows1_v, ss1)
        wait_sc(j + 1, rows3_v, ss3)
        gather(j + 5, rows3_v, sem3)
        return c

      lax.fori_loop(0, 5, body, 0)
      wait_gather(22, rows2_v, sem2)
      fire_sc(22, rows2_v, ss2)
      wait_sc(20, rows0_v, ss0)
      gather(24, rows0_v, sem0)
      wait_gather(23, rows3_v, sem3)
      fire_sc(23, rows3_v, ss3)
      wait_sc(21, rows1_v, ss1)
      wait_gather(24, rows0_v, sem0)
      fire_sc(24, rows0_v, ss0)
      wait_sc(22, rows2_v, ss2)
      wait_sc(23, rows3_v, ss3)
      wait_sc(24, rows0_v, ss0)
      return carry

    lax.fori_loop(0, NSC, superchunk, 0)
    plsc.subcore_barrier()
    pltpu.sync_copy(agg_s.at[pl.ds(sid * RPT, RPT)],
                    out_hbm.at[cid, pl.ds(sid * RPT, RPT)])

    @pl.when(sid == 15)
    def _out_tail():
      pltpu.sync_copy(agg_s.at[pl.ds(TAIL_OFF, TAIL)],
                      out_hbm.at[cid, pl.ds(TAIL_OFF, TAIL)])

  return k(h, src3, dst3, zeros)


def _bn_cols(z, gamma, beta):
  mu = jnp.mean(z, axis=0, keepdims=True)
  var = jnp.mean((z - mu) * (z - mu), axis=0, keepdims=True)
  return gamma * (z - mu) / jnp.sqrt(var + 1e-5) + beta


def _mlp_mid(scale_ref, h_ref, a0_ref, a1_ref, w1_ref, b1_ref, g1_ref,
             be1_ref, w2_ref, b2_ref, go_ref, bo_ref, out_ref):
  z = scale_ref[0, 0] * h_ref[...] + a0_ref[...] + a1_ref[...]
  z = jnp.dot(z, w1_ref[...], preferred_element_type=jnp.float32) + b1_ref[...]
  z = _bn_cols(z, g1_ref[...], be1_ref[...])
  z = jnp.maximum(z, 0.0)
  z = jnp.dot(z, w2_ref[...], preferred_element_type=jnp.float32) + b2_ref[...]
  z = _bn_cols(z, go_ref[...], bo_ref[...])
  out_ref[...] = jnp.maximum(z, 0.0)


def _mlp_last(scale_ref, h_ref, a0_ref, a1_ref, w1_ref, b1_ref, g1_ref,
              be1_ref, w2_ref, b2_ref, out_ref):
  z = scale_ref[0, 0] * h_ref[...] + a0_ref[...] + a1_ref[...]
  z = jnp.dot(z, w1_ref[...], preferred_element_type=jnp.float32) + b1_ref[...]
  z = _bn_cols(z, g1_ref[...], be1_ref[...])
  z = jnp.maximum(z, 0.0)
  z = jnp.dot(z, w2_ref[...], preferred_element_type=jnp.float32) + b2_ref[...]
  m = jnp.max(z, axis=-1, keepdims=True)
  s = z - m
  out_ref[...] = s - jnp.log(jnp.sum(jnp.exp(s), axis=-1, keepdims=True))


def _tc_mlp(scale, h, a0, a1, *weights, last):
  body = _mlp_last if last else _mlp_mid
  n_vmem = 3 + len(weights)
  return pl.pallas_call(
      body,
      out_shape=jax.ShapeDtypeStruct((N, F), jnp.float32),
      in_specs=[pl.BlockSpec(memory_space=pltpu.SMEM)]
      + [pl.BlockSpec(memory_space=pltpu.VMEM)] * n_vmem,
      out_specs=pl.BlockSpec(memory_space=pltpu.VMEM),
  )(scale, h, a0, a1, *weights)


def kernel(x, edge_index, eps, W1, b1, g1, be1, W2, b2, go, bo):
  src3 = edge_index[0].reshape(NW, NSC, SCH, CH)
  dst3 = edge_index[1].reshape(NW, NSC, SCH, CH)
  zeros = jnp.zeros((N, F), jnp.float32)
  h = x
  for l in range(3):
    parts = _sc_segment_sum(h, src3, dst3, zeros)
    scale = (1.0 + eps[l]).reshape(1, 1)
    row = lambda v: v.reshape(1, -1)
    if l < 2:
      h = _tc_mlp(scale, h, parts[0], parts[1], W1[l], row(b1[l]),
                  row(g1[l]), row(be1[l]), W2[l], row(b2[l]), row(go[l]),
                  row(bo[l]), last=False)
    else:
      h = _tc_mlp(scale, h, parts[0], parts[1], W1[l], row(b1[l]),
                  row(g1[l]), row(be1[l]), W2[l], row(b2[l]), last=True)
  return h


# speedup vs baseline: 1.1123x; 1.1123x over previous
import functools

import jax
import jax.numpy as jnp
from jax import lax
from jax.experimental import pallas as pl
from jax.experimental.pallas import tpu as pltpu
from jax.experimental.pallas import tpu_sc as plsc

N = 10000
F = 128
E = 320000
NW = 32
EPW = E // NW
CH = 80
NCH = EPW // CH
SCH = 25
NSC = NCH // SCH
RPT = 624
TAIL_OFF = RPT * 16
TAIL = N - TAIL_OFF


def _sc_segment_sum(h, src3, dst3, zeros):
  mesh = plsc.VectorSubcoreMesh(core_axis_name="c", subcore_axis_name="s")

  @functools.partial(
      pl.kernel,
      out_type=jax.ShapeDtypeStruct((2, N, F), jnp.float32),
      mesh=mesh,
      scratch_types=[
          pltpu.VMEM((SCH, CH), jnp.int32),
          pltpu.VMEM((SCH, CH), jnp.int32),
          pltpu.VMEM((CH, F), jnp.float32),
          pltpu.VMEM((CH, F), jnp.float32),
          pltpu.VMEM((CH, F), jnp.float32),
          pltpu.VMEM((CH, F), jnp.float32),
          pltpu.VMEM_SHARED((N, F), jnp.float32),
          pltpu.SemaphoreType.DMA,
          pltpu.SemaphoreType.DMA,
          pltpu.SemaphoreType.DMA,
          pltpu.SemaphoreType.DMA,
      ],
  )
  def k(h_hbm, src_hbm, dst_hbm, z_hbm, out_hbm, src_v, dst_v, rows0_v,
        rows1_v, rows2_v, rows3_v, agg_s, sem0, sem1, sem2, sem3):
    cid = lax.axis_index("c")
    sid = lax.axis_index("s")
    wid = cid * 16 + sid
    pltpu.sync_copy(z_hbm.at[pl.ds(sid * RPT, RPT)],
                    agg_s.at[pl.ds(sid * RPT, RPT)])

    @pl.when(sid == 15)
    def _zero_tail():
      pltpu.sync_copy(z_hbm.at[pl.ds(TAIL_OFF, TAIL)],
                      agg_s.at[pl.ds(TAIL_OFF, TAIL)])

    plsc.subcore_barrier()

    def gather(j, buf, sem):
      return pltpu.async_copy(h_hbm.at[src_v.at[j]], buf, sem)

    def wait_gather(j, buf, sem):
      pltpu.make_async_copy(h_hbm.at[src_v.at[j]], buf, sem).wait()

    def scatter(j, buf):
      pltpu.sync_copy(buf, agg_s.at[dst_v.at[j]], add=True)

    def superchunk(s, carry):
      pltpu.sync_copy(src_hbm.at[wid, s], src_v)
      pltpu.sync_copy(dst_hbm.at[wid, s], dst_v)
      gather(0, rows0_v, sem0)
      gather(1, rows1_v, sem1)
      gather(2, rows2_v, sem2)

      def body(i, c):
        j = 4 * i
        gather(j + 3, rows3_v, sem3)
        wait_gather(j, rows0_v, sem0)
        scatter(j, rows0_v)
        gather(j + 4, rows0_v, sem0)
        wait_gather(j + 1, rows1_v, sem1)
        scatter(j + 1, rows1_v)
        gather(j + 5, rows1_v, sem1)
        wait_gather(j + 2, rows2_v, sem2)
        scatter(j + 2, rows2_v)
        gather(j + 6, rows2_v, sem2)
        wait_gather(j + 3, rows3_v, sem3)
        scatter(j + 3, rows3_v)
        return c

      lax.fori_loop(0, 5, body, 0)
      gather(23, rows3_v, sem3)
      wait_gather(20, rows0_v, sem0)
      scatter(20, rows0_v)
      gather(24, rows0_v, sem0)
      wait_gather(21, rows1_v, sem1)
      scatter(21, rows1_v)
      wait_gather(22, rows2_v, sem2)
      scatter(22, rows2_v)
      wait_gather(23, rows3_v, sem3)
      scatter(23, rows3_v)
      wait_gather(24, rows0_v, sem0)
      scatter(24, rows0_v)
      return carry

    lax.fori_loop(0, NSC, superchunk, 0)
    plsc.subcore_barrier()
    pltpu.sync_copy(agg_s.at[pl.ds(sid * RPT, RPT)],
                    out_hbm.at[cid, pl.ds(sid * RPT, RPT)])

    @pl.when(sid == 15)
    def _out_tail():
      pltpu.sync_copy(agg_s.at[pl.ds(TAIL_OFF, TAIL)],
                      out_hbm.at[cid, pl.ds(TAIL_OFF, TAIL)])

  return k(h, src3, dst3, zeros)


def _bn_cols(z, gamma, beta):
  mu = jnp.mean(z, axis=0, keepdims=True)
  var = jnp.mean((z - mu) * (z - mu), axis=0, keepdims=True)
  return gamma * (z - mu) / jnp.sqrt(var + 1e-5) + beta


def _mlp_mid(scale_ref, h_ref, a0_ref, a1_ref, w1_ref, b1_ref, g1_ref,
             be1_ref, w2_ref, b2_ref, go_ref, bo_ref, out_ref):
  z = scale_ref[0, 0] * h_ref[...] + a0_ref[...] + a1_ref[...]
  z = jnp.dot(z, w1_ref[...], preferred_element_type=jnp.float32) + b1_ref[...]
  z = _bn_cols(z, g1_ref[...], be1_ref[...])
  z = jnp.maximum(z, 0.0)
  z = jnp.dot(z, w2_ref[...], preferred_element_type=jnp.float32) + b2_ref[...]
  z = _bn_cols(z, go_ref[...], bo_ref[...])
  out_ref[...] = jnp.maximum(z, 0.0)


def _mlp_last(scale_ref, h_ref, a0_ref, a1_ref, w1_ref, b1_ref, g1_ref,
              be1_ref, w2_ref, b2_ref, out_ref):
  z = scale_ref[0, 0] * h_ref[...] + a0_ref[...] + a1_ref[...]
  z = jnp.dot(z, w1_ref[...], preferred_element_type=jnp.float32) + b1_ref[...]
  z = _bn_cols(z, g1_ref[...], be1_ref[...])
  z = jnp.maximum(z, 0.0)
  z = jnp.dot(z, w2_ref[...], preferred_element_type=jnp.float32) + b2_ref[...]
  m = jnp.max(z, axis=-1, keepdims=True)
  s = z - m
  out_ref[...] = s - jnp.log(jnp.sum(jnp.exp(s), axis=-1, keepdims=True))


def _tc_mlp(scale, h, a0, a1, *weights, last):
  body = _mlp_last if last else _mlp_mid
  n_vmem = 3 + len(weights)
  return pl.pallas_call(
      body,
      out_shape=jax.ShapeDtypeStruct((N, F), jnp.float32),
      in_specs=[pl.BlockSpec(memory_space=pltpu.SMEM)]
      + [pl.BlockSpec(memory_space=pltpu.VMEM)] * n_vmem,
      out_specs=pl.BlockSpec(memory_space=pltpu.VMEM),
  )(scale, h, a0, a1, *weights)


def kernel(x, edge_index, eps, W1, b1, g1, be1, W2, b2, go, bo):
  src3 = edge_index[0].reshape(NW, NSC, SCH, CH)
  dst3 = edge_index[1].reshape(NW, NSC, SCH, CH)
  zeros = jnp.zeros((N, F), jnp.float32)
  h = x
  for l in range(3):
    parts = _sc_segment_sum(h, src3, dst3, zeros)
    scale = (1.0 + eps[l]).reshape(1, 1)
    row = lambda v: v.reshape(1, -1)
    if l < 2:
      h = _tc_mlp(scale, h, parts[0], parts[1], W1[l], row(b1[l]),
                  row(g1[l]), row(be1[l]), W2[l], row(b2[l]), row(go[l]),
                  row(bo[l]), last=False)
    else:
      h = _tc_mlp(scale, h, parts[0], parts[1], W1[l], row(b1[l]),
                  row(g1[l]), row(be1[l]), W2[l], row(b2[l]), last=True)
  return h
